# Initial kernel scaffold; baseline (speedup 1.0000x reference)
import jax
import jax.numpy as jnp
from jax.experimental import pallas as pl


def kernel(input_tensor, meter_table, unit_table, W1, b1, W2, b2):
    raise NotImplementedError("write your pallas kernel here")


# speedup vs baseline: 4.1900x; 4.1900x over previous
import jax
import jax.numpy as jnp
from jax.experimental import pallas as pl

_VPAD = 128
_EMB = 128
_HID2 = 512
_OUT = 256


def _prep_kernel(tables_ref, w1at_ref, w1bt_ref, ab_ref):
    ab_ref[0:_VPAD, :] = jnp.dot(
        tables_ref[0:_VPAD, :], w1at_ref[...], preferred_element_type=jnp.float32
    )
    ab_ref[_VPAD : 2 * _VPAD, :] = jnp.dot(
        tables_ref[_VPAD : 2 * _VPAD, :], w1bt_ref[...],
        preferred_element_type=jnp.float32,
    )


def _main_kernel(idx_ref, ab_ref, b1_ref, w2t_ref, b2_ref, out_ref):
    bn = idx_ref.shape[0]
    idx2 = idx_ref[:, 2:3]
    idx3 = idx_ref[:, 3:4] + _VPAD
    iota = jax.lax.broadcasted_iota(jnp.int32, (bn, 2 * _VPAD), 1)
    oh = ((iota == idx2) | (iota == idx3)).astype(jnp.float32)
    h = jnp.dot(oh, ab_ref[...], preferred_element_type=jnp.float32)
    h = jnp.maximum(h + b1_ref[...], 0.0)
    out_ref[...] = (
        jnp.dot(h, w2t_ref[...], preferred_element_type=jnp.float32) + b2_ref[...]
    )


def kernel(input_tensor, meter_table, unit_table, W1, b1, W2, b2):
    n = input_tensor.shape[0]
    bn = 2048
    meter_pad = jnp.pad(meter_table, ((0, _VPAD - meter_table.shape[0]), (0, 0)))
    unit_pad = jnp.pad(unit_table, ((0, _VPAD - unit_table.shape[0]), (0, 0)))
    tables = jnp.concatenate([meter_pad, unit_pad], axis=0)
    w1at = W1[:, :_EMB].T
    w1bt = W1[:, _EMB:].T
    w2t = W2.T

    ab = pl.pallas_call(
        _prep_kernel,
        out_shape=jax.ShapeDtypeStruct((2 * _VPAD, _HID2), jnp.float32),
    )(tables, w1at, w1bt)

    out = pl.pallas_call(
        _main_kernel,
        grid=(n // bn,),
        in_specs=[
            pl.BlockSpec((bn, 4), lambda i: (i, 0)),
            pl.BlockSpec((2 * _VPAD, _HID2), lambda i: (0, 0)),
            pl.BlockSpec((1, _HID2), lambda i: (0, 0)),
            pl.BlockSpec((_HID2, _OUT), lambda i: (0, 0)),
            pl.BlockSpec((1, _OUT), lambda i: (0, 0)),
        ],
        out_specs=pl.BlockSpec((bn, _OUT), lambda i: (i, 0)),
        out_shape=jax.ShapeDtypeStruct((n, _OUT), jnp.float32),
    )(input_tensor, ab, b1.reshape(1, _HID2), w2t, b2.reshape(1, _OUT))
    return out
